# Initial kernel scaffold; baseline (speedup 1.0000x reference)
import jax
import jax.numpy as jnp
from jax.experimental import pallas as pl


def kernel(s):
    raise NotImplementedError("write your pallas kernel here")


# speedup vs baseline: 4.4396x; 4.4396x over previous
import functools

import jax
import jax.numpy as jnp
import numpy as np
from jax import lax
from jax.experimental import pallas as pl
from jax.experimental.pallas import tpu as pltpu
from jax.experimental.pallas import tpu_sc as plsc

N = 4096
K = 128
LANES = 16
NC, NS = 2, 16
NW = NC * NS
ROWS_PER_W = N // NW
VPR = N // LANES
UNROLL = 8

_SIGN = np.uint32(0x80000000)
_ONE_U = np.uint32(1)


def _sc_body(s_hbm, tot_hbm, diag_hbm, row_v, u_v, hist, cbuf,
             tot_res, diag_res):
    wid = lax.axis_index("s") * NC + lax.axis_index("c")
    row0 = wid * ROWS_PER_W
    lane_iota = lax.iota(jnp.int32, LANES)
    lane0 = lane_iota == 0
    ones_i = jnp.ones((LANES,), jnp.int32)
    zeros_i = jnp.zeros((LANES,), jnp.int32)

    def hist_zero():
        for j in range(256 // LANES):
            hist[pl.ds(j * LANES, LANES)] = zeros_i

    def hist_pick(kk):
        def scan_step(j, carry):
            carryc, B, cnt_gt, found = carry
            jj = 15 - j
            h = hist[pl.ds(jj * LANES, LANES)]
            hd = lax.rev(h, (0,))
            csum = plsc.cumsum(hd) + carryc
            m = csum >= kk
            lane = jnp.max(plsc.all_reduce_ffs(m))
            sel = lane_iota == lane
            S_B = jnp.sum(jnp.where(sel, csum, 0))
            h_B = jnp.sum(jnp.where(sel, hd, 0))
            found_now = jnp.any(m) & jnp.logical_not(found)
            B = jnp.where(found_now, jj * LANES + 15 - lane, B)
            cnt_gt = jnp.where(found_now, S_B - h_B, cnt_gt)
            return (jnp.max(csum), B, cnt_gt, found | jnp.any(m))

        _, B, cnt_gt, _ = lax.fori_loop(
            0, 256 // LANES, scan_step,
            (jnp.int32(0), jnp.int32(0), jnp.int32(0), False))
        return B, cnt_gt

    def do_row(r, _):
        grow = row0 + r
        pltpu.sync_copy(s_hbm.at[grow], row_v)

        hist_zero()

        def prep_step(j, carry):
            dmax = carry
            for k in range(UNROLL):
                base = (j * UNROLL + k) * LANES
                v = row_v[pl.ds(base, LANES)]
                col = lane_iota + base
                isdiag = col == grow
                dmax = jnp.maximum(dmax, jnp.where(isdiag, v, -3.4e38))
                b = lax.bitcast_convert_type(v, jnp.uint32)
                u = jnp.where(b >= _SIGN, ~b, b | _SIGN)
                u = jnp.where(isdiag, jnp.uint32(0), u)
                u_v[pl.ds(base, LANES)] = u
                idx = lax.shift_right_logical(u, jnp.uint32(24))
                plsc.addupdate_scatter(hist, [idx.astype(jnp.int32)], ones_i)
            return dmax

        dmax = lax.fori_loop(
            0, VPR // UNROLL, prep_step,
            jnp.full((LANES,), -3.4e38, jnp.float32), unroll=False)
        diag = jnp.max(dmax)

        B1, cnt_gt1 = hist_pick(jnp.int32(K))
        k1 = K - cnt_gt1
        B1u = B1.astype(jnp.uint32)

        hist_zero()

        def h2_step(j, _c):
            for k in range(UNROLL):
                base = (j * UNROLL + k) * LANES
                u = u_v[pl.ds(base, LANES)]
                m = lax.shift_right_logical(u, jnp.uint32(24)) == B1u
                idx = lax.shift_right_logical(u, jnp.uint32(16)) \
                    & jnp.uint32(0xFF)
                plsc.addupdate_scatter(hist, [idx.astype(jnp.int32)], ones_i,
                                       mask=m)
            return _c

        lax.fori_loop(0, VPR // UNROLL, h2_step, 0, unroll=False)
        B2, cnt_gt2 = hist_pick(k1)
        c_hi = cnt_gt1 + cnt_gt2
        lo = lax.shift_left(B1u * jnp.uint32(256) + B2.astype(jnp.uint32),
                            jnp.uint32(16))
        bmax = lo | jnp.uint32(0xFFFF)

        def p3_step(j, carry):
            acc_s, off = carry
            for k in range(UNROLL):
                base = (j * UNROLL + k) * LANES
                u = u_v[pl.ds(base, LANES)]
                m_hi = u > bmax
                bits = jnp.where(u >= _SIGN, u & ~_SIGN, ~u)
                e = jnp.exp(lax.bitcast_convert_type(bits, jnp.float32))
                acc_s = acc_s + jnp.where(m_hi, e, 0.0)
                m_band = jnp.logical_and(u >= lo, u <= bmax)
                plsc.store_compressed(cbuf.at[pl.ds(off, LANES)], u,
                                      mask=m_band)
                off = off + jnp.max(
                    plsc.all_reduce_population_count(m_band))
            return acc_s, off

        acc_hi, nband = lax.fori_loop(
            0, VPR // UNROLL, p3_step,
            (jnp.zeros((LANES,), jnp.float32), jnp.int32(0)), unroll=False)
        hi_sum = jnp.sum(acc_hi)
        cbuf[pl.ds(nband, LANES)] = jnp.zeros((LANES,), jnp.uint32)
        nv = (nband + LANES - 1) // LANES

        def bit_step(i, T):
            t_try = T | lax.shift_left(_ONE_U, jnp.uint32(15 - i))

            def cnt_step(j, acc):
                u = cbuf[pl.ds(j * LANES, LANES)]
                return acc + jnp.where(u >= t_try, 1, 0).astype(jnp.int32)

            cnt = jnp.sum(lax.fori_loop(0, nv, cnt_step, zeros_i))
            return jnp.where(c_hi + cnt >= K, t_try, T)

        T = lax.fori_loop(0, 16, bit_step, lo, unroll=False)

        def fin_step(j, carry):
            acc_s, acc_c = carry
            u = cbuf[pl.ds(j * LANES, LANES)]
            m = u > T
            bits = jnp.where(u >= _SIGN, u & ~_SIGN, ~u)
            e = jnp.exp(lax.bitcast_convert_type(bits, jnp.float32))
            acc_s = acc_s + jnp.where(m, e, 0.0)
            acc_c = acc_c + jnp.where(m, 1, 0).astype(jnp.int32)
            return acc_s, acc_c

        acc_s, acc_c = lax.fori_loop(
            0, nv, fin_step,
            (jnp.zeros((LANES,), jnp.float32), zeros_i))
        band_sum = jnp.sum(acc_s)
        c_strict = c_hi + jnp.sum(acc_c)

        t_bits = jnp.where(T >= _SIGN, T & ~_SIGN, ~T)
        t_val = jnp.max(lax.bitcast_convert_type(jnp.full((LANES,), t_bits),
                                                 jnp.float32))
        pair = jnp.where(lane0, diag, t_val)
        epair = jnp.exp(pair)
        exp_diag = jnp.max(jnp.where(lane0, epair, -1.0))
        exp_t = jnp.max(jnp.where(lane0, -1.0, epair))

        total = (hi_sum + band_sum
                 + (K - c_strict).astype(jnp.float32) * exp_t + exp_diag)
        plsc.store_scatter(tot_res, [jnp.full((LANES,), r, jnp.int32)],
                           jnp.full((LANES,), total), mask=lane0)
        plsc.store_scatter(diag_res, [jnp.full((LANES,), r, jnp.int32)],
                           jnp.full((LANES,), diag), mask=lane0)
        return 0

    lax.fori_loop(0, ROWS_PER_W, do_row, 0, unroll=False)
    pltpu.sync_copy(tot_res, tot_hbm.at[pl.ds(row0, ROWS_PER_W)])
    pltpu.sync_copy(diag_res, diag_hbm.at[pl.ds(row0, ROWS_PER_W)])


@jax.jit
def _sc_select(s):
    mesh = plsc.VectorSubcoreMesh(core_axis_name="c", subcore_axis_name="s",
                                  num_cores=NC, num_subcores=NS)
    return pl.kernel(
        _sc_body,
        out_type=[
            jax.ShapeDtypeStruct((N,), jnp.float32),
            jax.ShapeDtypeStruct((N,), jnp.float32),
        ],
        mesh=mesh,
        compiler_params=pltpu.CompilerParams(needs_layout_passes=False),
        scratch_types=[
            pltpu.VMEM((N,), jnp.float32),
            pltpu.VMEM((N,), jnp.uint32),
            pltpu.VMEM((256,), jnp.int32),
            pltpu.VMEM((N + 2 * LANES,), jnp.uint32),
            pltpu.VMEM((ROWS_PER_W,), jnp.float32),
            pltpu.VMEM((ROWS_PER_W,), jnp.float32),
        ],
    )(s)


def _finish_body(tot_ref, diag_ref, out_ref):
    out_ref[0, 0] = jnp.mean(jnp.log(tot_ref[...]) - diag_ref[...])


@jax.jit
def _tc_finish(tot, diag):
    return pl.pallas_call(
        _finish_body,
        out_shape=jax.ShapeDtypeStruct((1, 1), jnp.float32),
        out_specs=pl.BlockSpec(memory_space=pltpu.SMEM),
    )(tot, diag)


def kernel(s):
    tot, diag = _sc_select(s)
    out = _tc_finish(tot.reshape(32, ROWS_PER_W), diag.reshape(32, ROWS_PER_W))
    return out[0, 0]


# speedup vs baseline: 4.9293x; 1.1103x over previous
import functools

import jax
import jax.numpy as jnp
import numpy as np
from jax import lax
from jax.experimental import pallas as pl
from jax.experimental.pallas import tpu as pltpu
from jax.experimental.pallas import tpu_sc as plsc

N = 4096
K = 128
LANES = 16
NC, NS = 2, 16
NW = NC * NS
ROWS_PER_W = N // NW
VPR = N // LANES
UNROLL = 8
HI_BITS = 10
LO_BITS = 32 - HI_BITS

_SIGN = np.uint32(0x80000000)
_ONE_U = np.uint32(1)
_BAND_MASK = np.uint32((1 << LO_BITS) - 1)


def _sc_body(s_hbm, tot_hbm, diag_hbm, row_v, u_v, cbuf, tot_res, diag_res):
    wid = lax.axis_index("s") * NC + lax.axis_index("c")
    row0 = wid * ROWS_PER_W
    lane_iota = lax.iota(jnp.int32, LANES)
    lane0 = lane_iota == 0
    zeros_i = jnp.zeros((LANES,), jnp.int32)

    def do_row(r, _):
        grow = row0 + r
        pltpu.sync_copy(s_hbm.at[grow], row_v)

        def prep_step(j, carry):
            dmax = carry
            for k in range(UNROLL):
                base = (j * UNROLL + k) * LANES
                v = row_v[pl.ds(base, LANES)]
                col = lane_iota + base
                isdiag = col == grow
                dmax = jnp.maximum(dmax, jnp.where(isdiag, v, -3.4e38))
                b = lax.bitcast_convert_type(v, jnp.uint32)
                u = jnp.where(b >= _SIGN, ~b, b | _SIGN)
                u = jnp.where(isdiag, jnp.uint32(0), u)
                u_v[pl.ds(base, LANES)] = u
            return dmax

        dmax = lax.fori_loop(
            0, VPR // UNROLL, prep_step,
            jnp.full((LANES,), -3.4e38, jnp.float32), unroll=False)
        diag = jnp.max(dmax)

        def bit_step(i, T):
            t_try = T | lax.shift_left(_ONE_U, jnp.uint32(31 - i))

            def cnt_step(j, acc):
                for k in range(UNROLL):
                    base = (j * UNROLL + k) * LANES
                    u = u_v[pl.ds(base, LANES)]
                    acc = acc + jnp.where(u >= t_try, 1, 0).astype(jnp.int32)
                return acc

            cnt = jnp.sum(lax.fori_loop(
                0, VPR // UNROLL, cnt_step, zeros_i, unroll=False))
            return jnp.where(cnt >= K, t_try, T)

        T = lax.fori_loop(0, HI_BITS, bit_step, jnp.uint32(0), unroll=False)
        lo = T
        bmax = T | _BAND_MASK

        def p3_step(j, carry):
            acc_s, acc_c, off = carry
            for k in range(UNROLL):
                base = (j * UNROLL + k) * LANES
                u = u_v[pl.ds(base, LANES)]
                m_hi = u > bmax
                bits = jnp.where(u >= _SIGN, u & ~_SIGN, ~u)
                e = jnp.exp(lax.bitcast_convert_type(bits, jnp.float32))
                acc_s = acc_s + jnp.where(m_hi, e, 0.0)
                acc_c = acc_c + jnp.where(m_hi, 1, 0).astype(jnp.int32)
                m_band = jnp.logical_and(u >= lo, u <= bmax)
                plsc.store_compressed(cbuf.at[pl.ds(off, LANES)], u,
                                      mask=m_band)
                off = off + jnp.max(
                    plsc.all_reduce_population_count(m_band))
            return acc_s, acc_c, off

        acc_hi, acc_chi, nband = lax.fori_loop(
            0, VPR // UNROLL, p3_step,
            (jnp.zeros((LANES,), jnp.float32), zeros_i, jnp.int32(0)),
            unroll=False)
        hi_sum = jnp.sum(acc_hi)
        c_hi = jnp.sum(acc_chi)
        cbuf[pl.ds(nband, LANES)] = jnp.zeros((LANES,), jnp.uint32)
        nv = (nband + LANES - 1) // LANES

        def lo_bit_step(i, T):
            t_try = T | lax.shift_left(_ONE_U, jnp.uint32(LO_BITS - 1 - i))

            def cnt_step(j, acc):
                u = cbuf[pl.ds(j * LANES, LANES)]
                return acc + jnp.where(u >= t_try, 1, 0).astype(jnp.int32)

            cnt = jnp.sum(lax.fori_loop(0, nv, cnt_step, zeros_i))
            return jnp.where(c_hi + cnt >= K, t_try, T)

        T = lax.fori_loop(0, LO_BITS, lo_bit_step, lo, unroll=False)

        def fin_step(j, carry):
            acc_s, acc_c = carry
            u = cbuf[pl.ds(j * LANES, LANES)]
            m = u > T
            bits = jnp.where(u >= _SIGN, u & ~_SIGN, ~u)
            e = jnp.exp(lax.bitcast_convert_type(bits, jnp.float32))
            acc_s = acc_s + jnp.where(m, e, 0.0)
            acc_c = acc_c + jnp.where(m, 1, 0).astype(jnp.int32)
            return acc_s, acc_c

        acc_s, acc_c = lax.fori_loop(
            0, nv, fin_step,
            (jnp.zeros((LANES,), jnp.float32), zeros_i))
        band_sum = jnp.sum(acc_s)
        c_strict = c_hi + jnp.sum(acc_c)

        t_bits = jnp.where(T >= _SIGN, T & ~_SIGN, ~T)
        t_val = jnp.max(lax.bitcast_convert_type(jnp.full((LANES,), t_bits),
                                                 jnp.float32))
        pair = jnp.where(lane0, diag, t_val)
        epair = jnp.exp(pair)
        exp_diag = jnp.max(jnp.where(lane0, epair, -1.0))
        exp_t = jnp.max(jnp.where(lane0, -1.0, epair))

        total = (hi_sum + band_sum
                 + (K - c_strict).astype(jnp.float32) * exp_t + exp_diag)
        plsc.store_scatter(tot_res, [jnp.full((LANES,), r, jnp.int32)],
                           jnp.full((LANES,), total), mask=lane0)
        plsc.store_scatter(diag_res, [jnp.full((LANES,), r, jnp.int32)],
                           jnp.full((LANES,), diag), mask=lane0)
        return 0

    lax.fori_loop(0, ROWS_PER_W, do_row, 0, unroll=False)
    pltpu.sync_copy(tot_res, tot_hbm.at[pl.ds(row0, ROWS_PER_W)])
    pltpu.sync_copy(diag_res, diag_hbm.at[pl.ds(row0, ROWS_PER_W)])


@jax.jit
def _sc_select(s):
    mesh = plsc.VectorSubcoreMesh(core_axis_name="c", subcore_axis_name="s",
                                  num_cores=NC, num_subcores=NS)
    return pl.kernel(
        _sc_body,
        out_type=[
            jax.ShapeDtypeStruct((N,), jnp.float32),
            jax.ShapeDtypeStruct((N,), jnp.float32),
        ],
        mesh=mesh,
        compiler_params=pltpu.CompilerParams(needs_layout_passes=False),
        scratch_types=[
            pltpu.VMEM((N,), jnp.float32),
            pltpu.VMEM((N,), jnp.uint32),
            pltpu.VMEM((N + 2 * LANES,), jnp.uint32),
            pltpu.VMEM((ROWS_PER_W,), jnp.float32),
            pltpu.VMEM((ROWS_PER_W,), jnp.float32),
        ],
    )(s)


def _finish_body(tot_ref, diag_ref, out_ref):
    out_ref[0, 0] = jnp.mean(jnp.log(tot_ref[...]) - diag_ref[...])


@jax.jit
def _tc_finish(tot, diag):
    return pl.pallas_call(
        _finish_body,
        out_shape=jax.ShapeDtypeStruct((1, 1), jnp.float32),
        out_specs=pl.BlockSpec(memory_space=pltpu.SMEM),
    )(tot, diag)


def kernel(s):
    tot, diag = _sc_select(s)
    out = _tc_finish(tot.reshape(32, ROWS_PER_W), diag.reshape(32, ROWS_PER_W))
    return out[0, 0]


# speedup vs baseline: 5.3124x; 1.0777x over previous
import functools

import jax
import jax.numpy as jnp
import numpy as np
from jax import lax
from jax.experimental import pallas as pl
from jax.experimental.pallas import tpu as pltpu
from jax.experimental.pallas import tpu_sc as plsc

N = 4096
K = 128
LANES = 16
NC, NS = 2, 16
NW = NC * NS
ROWS_PER_W = N // NW
VPR = N // LANES
UNROLL = 8
HI_BITS = 10
LO_BITS = 32 - HI_BITS

_SIGN = np.uint32(0x80000000)
_ONE_U = np.uint32(1)
_BAND_MASK = np.uint32((1 << LO_BITS) - 1)


def _sc_body(s_hbm, tot_hbm, diag_hbm, row_v, u_v, cbuf, tot_res, diag_res):
    wid = lax.axis_index("s") * NC + lax.axis_index("c")
    row0 = wid * ROWS_PER_W
    lane_iota = lax.iota(jnp.int32, LANES)
    lane0 = lane_iota == 0
    zeros_i = jnp.zeros((LANES,), jnp.int32)

    def do_row(r, _):
        grow = row0 + r
        pltpu.sync_copy(s_hbm.at[grow], row_v)

        def prep_step(j, carry):
            dmax = carry
            for k in range(UNROLL):
                base = (j * UNROLL + k) * LANES
                v = row_v[pl.ds(base, LANES)]
                col = lane_iota + base
                isdiag = col == grow
                dmax = jnp.maximum(dmax, jnp.where(isdiag, v, -3.4e38))
                b = lax.bitcast_convert_type(v, jnp.uint32)
                u = jnp.where(b >= _SIGN, ~b, b | _SIGN)
                u = jnp.where(isdiag, jnp.uint32(0), u)
                u_v[pl.ds(base, LANES)] = u
            return dmax

        dmax = lax.fori_loop(
            0, VPR // UNROLL, prep_step,
            jnp.full((LANES,), -3.4e38, jnp.float32), unroll=False)
        diag = jnp.max(dmax)

        def bit_step(i, T):
            t_try = T | lax.shift_left(_ONE_U, jnp.uint32(31 - i))

            def cnt_step(j, acc):
                for k in range(UNROLL):
                    base = (j * UNROLL + k) * LANES
                    u = u_v[pl.ds(base, LANES)]
                    acc = acc + jnp.where(u >= t_try, 1, 0).astype(jnp.int32)
                return acc

            cnt = jnp.sum(lax.fori_loop(
                0, VPR // UNROLL, cnt_step, zeros_i, unroll=False))
            return jnp.where(cnt >= K, t_try, T)

        T = lax.fori_loop(0, HI_BITS, bit_step, jnp.uint32(0), unroll=False)
        lo = T
        bmax = T | _BAND_MASK

        def p3_step(j, carry):
            acc_s, acc_c, off = carry
            for k in range(UNROLL):
                base = (j * UNROLL + k) * LANES
                u = u_v[pl.ds(base, LANES)]
                m_hi = u > bmax
                bits = jnp.where(u >= _SIGN, u & ~_SIGN, ~u)
                e = jnp.exp(lax.bitcast_convert_type(bits, jnp.float32))
                acc_s = acc_s + jnp.where(m_hi, e, 0.0)
                acc_c = acc_c + jnp.where(m_hi, 1, 0).astype(jnp.int32)
                m_band = jnp.logical_and(u >= lo, u <= bmax)
                plsc.store_compressed(cbuf.at[pl.ds(off, LANES)], u,
                                      mask=m_band)
                off = off + plsc.all_reduce_population_count(m_band)[0]
            return acc_s, acc_c, off

        acc_hi, acc_chi, nband = lax.fori_loop(
            0, VPR // UNROLL, p3_step,
            (jnp.zeros((LANES,), jnp.float32), zeros_i, jnp.int32(0)),
            unroll=False)
        hi_sum = jnp.sum(acc_hi)
        c_hi = jnp.sum(acc_chi)
        cbuf[pl.ds(nband, LANES)] = jnp.zeros((LANES,), jnp.uint32)
        nv = (nband + LANES - 1) // LANES

        def lo_bit_step(i, T):
            t_try = T | lax.shift_left(_ONE_U, jnp.uint32(LO_BITS - 1 - i))

            def cnt_step(j, acc):
                u = cbuf[pl.ds(j * LANES, LANES)]
                return acc + jnp.where(u >= t_try, 1, 0).astype(jnp.int32)

            cnt = jnp.sum(lax.fori_loop(0, nv, cnt_step, zeros_i))
            return jnp.where(c_hi + cnt >= K, t_try, T)

        T = lax.fori_loop(0, LO_BITS, lo_bit_step, lo, unroll=False)

        def fin_step(j, carry):
            acc_s, acc_c = carry
            u = cbuf[pl.ds(j * LANES, LANES)]
            m = u > T
            bits = jnp.where(u >= _SIGN, u & ~_SIGN, ~u)
            e = jnp.exp(lax.bitcast_convert_type(bits, jnp.float32))
            acc_s = acc_s + jnp.where(m, e, 0.0)
            acc_c = acc_c + jnp.where(m, 1, 0).astype(jnp.int32)
            return acc_s, acc_c

        acc_s, acc_c = lax.fori_loop(
            0, nv, fin_step,
            (jnp.zeros((LANES,), jnp.float32), zeros_i))
        band_sum = jnp.sum(acc_s)
        c_strict = c_hi + jnp.sum(acc_c)

        t_bits = jnp.where(T >= _SIGN, T & ~_SIGN, ~T)
        t_val = jnp.max(lax.bitcast_convert_type(jnp.full((LANES,), t_bits),
                                                 jnp.float32))
        pair = jnp.where(lane0, diag, t_val)
        epair = jnp.exp(pair)
        exp_diag = jnp.max(jnp.where(lane0, epair, -1.0))
        exp_t = jnp.max(jnp.where(lane0, -1.0, epair))

        total = (hi_sum + band_sum
                 + (K - c_strict).astype(jnp.float32) * exp_t + exp_diag)
        plsc.store_scatter(tot_res, [jnp.full((LANES,), r, jnp.int32)],
                           jnp.full((LANES,), total), mask=lane0)
        plsc.store_scatter(diag_res, [jnp.full((LANES,), r, jnp.int32)],
                           jnp.full((LANES,), diag), mask=lane0)
        return 0

    lax.fori_loop(0, ROWS_PER_W, do_row, 0, unroll=False)
    pltpu.sync_copy(tot_res, tot_hbm.at[pl.ds(row0, ROWS_PER_W)])
    pltpu.sync_copy(diag_res, diag_hbm.at[pl.ds(row0, ROWS_PER_W)])


@jax.jit
def _sc_select(s):
    mesh = plsc.VectorSubcoreMesh(core_axis_name="c", subcore_axis_name="s",
                                  num_cores=NC, num_subcores=NS)
    return pl.kernel(
        _sc_body,
        out_type=[
            jax.ShapeDtypeStruct((N,), jnp.float32),
            jax.ShapeDtypeStruct((N,), jnp.float32),
        ],
        mesh=mesh,
        compiler_params=pltpu.CompilerParams(needs_layout_passes=False),
        scratch_types=[
            pltpu.VMEM((N,), jnp.float32),
            pltpu.VMEM((N,), jnp.uint32),
            pltpu.VMEM((N + 2 * LANES,), jnp.uint32),
            pltpu.VMEM((ROWS_PER_W,), jnp.float32),
            pltpu.VMEM((ROWS_PER_W,), jnp.float32),
        ],
    )(s)


def _finish_body(tot_ref, diag_ref, out_ref):
    out_ref[0, 0] = jnp.mean(jnp.log(tot_ref[...]) - diag_ref[...])


@jax.jit
def _tc_finish(tot, diag):
    return pl.pallas_call(
        _finish_body,
        out_shape=jax.ShapeDtypeStruct((1, 1), jnp.float32),
        out_specs=pl.BlockSpec(memory_space=pltpu.SMEM),
    )(tot, diag)


def kernel(s):
    tot, diag = _sc_select(s)
    out = _tc_finish(tot.reshape(32, ROWS_PER_W), diag.reshape(32, ROWS_PER_W))
    return out[0, 0]
